# baseline (speedup 1.0000x reference)
import functools

import jax
import jax.numpy as jnp
from jax import lax
from jax.experimental import pallas as pl
from jax.experimental.pallas import tpu as pltpu
from jax.experimental.pallas import tpu_sc as plsc

_B, _D, _T = 16, 256, 576
_K = 8192
_KB = 512
_NKB = _K // _KB
_GB = 4
_NBG = _B // _GB

_NW = 32
_BH = _B // 2
_RPW = (_BH * _T) // _NW
_CHUNK = 72
_NCHUNK = _RPW // _CHUNK


def _argmin_body(z_ref, emb_ref, idx_ref,
                 minv_ref, mini_ref, znorm_ref, en_ref):
    bg = pl.program_id(0)
    kb = pl.program_id(1)

    @pl.when(bg == 0)
    def _():
        emb = emb_ref[...]
        en_ref[kb] = jnp.sum(emb * emb, axis=1, keepdims=True)

    @pl.when(kb == 0)
    def _():
        for i in range(_GB):
            zb = z_ref[i]
            znorm_ref[i] = jnp.sum(zb * zb, axis=0, keepdims=True)

    kbase = (kb * _KB).astype(jnp.float32)
    enorm = en_ref[kb]
    emb2 = emb_ref[...] + emb_ref[...]
    for i in range(_GB):
        zb = z_ref[i]
        s2 = jax.lax.dot_general(emb2, zb, (((1,), (0,)), ((), ())),
                                 preferred_element_type=jnp.float32)
        d = (znorm_ref[i] + enorm) - s2
        bmin = jnp.min(d, axis=0, keepdims=True)
        bidxf = jnp.argmin(d, axis=0).astype(jnp.float32)[None, :]

        @pl.when(kb == 0)
        def _():
            minv_ref[i] = bmin
            mini_ref[i] = bidxf

        @pl.when(kb > 0)
        def _():
            better = bmin < minv_ref[i]
            minv_ref[i] = jnp.where(better, bmin, minv_ref[i])
            mini_ref[i] = jnp.where(better, bidxf + kbase, mini_ref[i])

    @pl.when(kb == _NKB - 1)
    def _():
        for i in range(_GB):
            idx_ref[i] = mini_ref[i].astype(jnp.int32)


_sc_mesh = plsc.VectorSubcoreMesh(core_axis_name="c", subcore_axis_name="s")


@functools.partial(
    pl.kernel,
    mesh=_sc_mesh,
    out_type=jax.ShapeDtypeStruct((_BH * _T, _D), jnp.float32),
    scratch_types=[
        pltpu.VMEM((_NCHUNK, _CHUNK), jnp.int32),
        pltpu.VMEM((_RPW, _D), jnp.float32),
        pltpu.SemaphoreType.DMA,
    ],
)
def _sc_gather(idx_hbm, table_hbm, out_hbm, idx_v, rows_v, sem):
    wid = lax.axis_index("s") * 2 + lax.axis_index("c")
    pltpu.sync_copy(idx_hbm.at[wid], idx_v)
    copies = [
        pltpu.async_copy(table_hbm.at[idx_v.at[j]],
                         rows_v.at[pl.ds(j * _CHUNK, _CHUNK)], sem)
        for j in range(_NCHUNK)
    ]
    for cp in copies:
        cp.wait()
    pltpu.sync_copy(rows_v, out_hbm.at[pl.ds(wid * _RPW, _RPW)])


def _transpose_body(in_ref, out_ref):
    out_ref[0] = in_ref[0].T


def _stage1(zh, embedding):
    return pl.pallas_call(
        _argmin_body,
        grid=(_BH // _GB, _NKB),
        in_specs=[pl.BlockSpec((_GB, _D, _T), lambda g, k: (g, 0, 0)),
                  pl.BlockSpec((_KB, _D), lambda g, k: (k, 0))],
        out_specs=pl.BlockSpec((_GB, 1, _T), lambda g, k: (g, 0, 0)),
        out_shape=jax.ShapeDtypeStruct((_BH, 1, _T), jnp.int32),
        scratch_shapes=[pltpu.VMEM((_GB, 1, _T), jnp.float32),
                        pltpu.VMEM((_GB, 1, _T), jnp.float32),
                        pltpu.VMEM((_GB, 1, _T), jnp.float32),
                        pltpu.VMEM((_NKB, _KB, 1), jnp.float32)],
    )(zh, embedding)


def _stage3(zq_flat):
    zq = zq_flat.reshape(_BH, _T, _D)
    return pl.pallas_call(
        _transpose_body,
        grid=(_BH,),
        in_specs=[pl.BlockSpec((1, _T, _D), lambda b: (b, 0, 0))],
        out_specs=pl.BlockSpec((1, _D, _T), lambda b: (b, 0, 0)),
        out_shape=jax.ShapeDtypeStruct((_BH, _D, _T), jnp.float32),
    )(zq)


def kernel(z, embedding):
    halves = []
    for h in range(2):
        zh = lax.slice_in_dim(z, h * _BH, (h + 1) * _BH, axis=0)
        idx = _stage1(zh, embedding)
        idx3 = idx.reshape(_NW, _NCHUNK, _CHUNK)
        halves.append(_sc_gather(idx3, embedding))
    out = jnp.concatenate([_stage3(g) for g in halves], axis=0)
    return out


# speedup vs baseline: 1.3834x; 1.3834x over previous
import functools

import jax
import jax.numpy as jnp
from jax import lax
from jax.experimental import pallas as pl
from jax.experimental.pallas import tpu as pltpu
from jax.experimental.pallas import tpu_sc as plsc

_B, _D, _T = 16, 256, 576
_K = 8192
_KB = 2048
_NKB = _K // _KB
_GB = 4
_NBG = _B // _GB

_NW = 32
_RPW = (_B * _T) // _NW
_CHUNK = 96
_NCHUNK = _RPW // _CHUNK


def _argmin_body(z_ref, emb_ref, idx_ref,
                 minv_ref, mini_ref, znorm_ref, en_ref):
    bg = pl.program_id(0)
    kb = pl.program_id(1)

    @pl.when(bg == 0)
    def _():
        emb = emb_ref[...]
        en_ref[kb] = jnp.sum(emb * emb, axis=1, keepdims=True)

    @pl.when(kb == 0)
    def _():
        for i in range(_GB):
            zb = z_ref[i]
            znorm_ref[i] = jnp.sum(zb * zb, axis=0, keepdims=True)

    kbase = (kb * _KB).astype(jnp.float32)
    enorm = en_ref[kb]
    emb2 = emb_ref[...] + emb_ref[...]
    for i in range(_GB):
        zb = z_ref[i]
        s2 = jax.lax.dot_general(emb2, zb, (((1,), (0,)), ((), ())),
                                 preferred_element_type=jnp.float32)
        d = (znorm_ref[i] + enorm) - s2
        bmin = jnp.min(d, axis=0, keepdims=True)
        bidxf = jnp.argmin(d, axis=0).astype(jnp.float32)[None, :]

        @pl.when(kb == 0)
        def _():
            minv_ref[i] = bmin
            mini_ref[i] = bidxf

        @pl.when(kb > 0)
        def _():
            better = bmin < minv_ref[i]
            minv_ref[i] = jnp.where(better, bmin, minv_ref[i])
            mini_ref[i] = jnp.where(better, bidxf + kbase, mini_ref[i])

    @pl.when(kb == _NKB - 1)
    def _():
        for i in range(_GB):
            idx_ref[i] = mini_ref[i].astype(jnp.int32)


_sc_mesh = plsc.VectorSubcoreMesh(core_axis_name="c", subcore_axis_name="s")


@functools.partial(
    pl.kernel,
    mesh=_sc_mesh,
    out_type=jax.ShapeDtypeStruct((_B * _T, _D), jnp.float32),
    scratch_types=[
        pltpu.VMEM((_NCHUNK, _CHUNK), jnp.int32),
        pltpu.VMEM((_RPW, _D), jnp.float32),
        pltpu.SemaphoreType.DMA,
    ],
)
def _sc_gather(idx_hbm, table_hbm, out_hbm, idx_v, rows_v, sem):
    wid = lax.axis_index("s") * 2 + lax.axis_index("c")
    pltpu.sync_copy(idx_hbm.at[wid], idx_v)
    copies = [
        pltpu.async_copy(table_hbm.at[idx_v.at[j]],
                         rows_v.at[pl.ds(j * _CHUNK, _CHUNK)], sem)
        for j in range(_NCHUNK)
    ]
    for cp in copies:
        cp.wait()
    pltpu.sync_copy(rows_v, out_hbm.at[pl.ds(wid * _RPW, _RPW)])


def _transpose_body(in_ref, out_ref):
    out_ref[0] = in_ref[0].T


def kernel(z, embedding):
    idx = pl.pallas_call(
        _argmin_body,
        grid=(_NBG, _NKB),
        in_specs=[pl.BlockSpec((_GB, _D, _T), lambda g, k: (g, 0, 0)),
                  pl.BlockSpec((_KB, _D), lambda g, k: (k, 0))],
        out_specs=pl.BlockSpec((_GB, 1, _T), lambda g, k: (g, 0, 0)),
        out_shape=jax.ShapeDtypeStruct((_B, 1, _T), jnp.int32),
        scratch_shapes=[pltpu.VMEM((_GB, 1, _T), jnp.float32),
                        pltpu.VMEM((_GB, 1, _T), jnp.float32),
                        pltpu.VMEM((_GB, 1, _T), jnp.float32),
                        pltpu.VMEM((_NKB, _KB, 1), jnp.float32)],
    )(z, embedding)
    idx3 = idx.reshape(_NW, _NCHUNK, _CHUNK)
    zq_flat = _sc_gather(idx3, embedding)
    zq = zq_flat.reshape(_B, _T, _D)
    out = pl.pallas_call(
        _transpose_body,
        grid=(_B,),
        in_specs=[pl.BlockSpec((1, _T, _D), lambda b: (b, 0, 0))],
        out_specs=pl.BlockSpec((1, _D, _T), lambda b: (b, 0, 0)),
        out_shape=jax.ShapeDtypeStruct((_B, _D, _T), jnp.float32),
    )(zq)
    return out


# speedup vs baseline: 1.3926x; 1.0066x over previous
import functools

import jax
import jax.numpy as jnp
from jax import lax
from jax.experimental import pallas as pl
from jax.experimental.pallas import tpu as pltpu
from jax.experimental.pallas import tpu_sc as plsc

_B, _D, _T = 16, 256, 576
_K = 8192
_KB = 2048
_NKB = _K // _KB
_GB = 8
_NBG = _B // _GB

_NW = 32
_RPW = (_B * _T) // _NW
_CHUNK = 96
_NCHUNK = _RPW // _CHUNK


def _argmin_body(z_ref, emb_ref, idx_ref,
                 minv_ref, mini_ref, znorm_ref, en_ref):
    bg = pl.program_id(0)
    kb = pl.program_id(1)

    @pl.when(bg == 0)
    def _():
        emb = emb_ref[...]
        en_ref[kb] = jnp.sum(emb * emb, axis=1, keepdims=True)

    @pl.when(kb == 0)
    def _():
        for i in range(_GB):
            zb = z_ref[i]
            znorm_ref[i] = jnp.sum(zb * zb, axis=0, keepdims=True)

    kbase = (kb * _KB).astype(jnp.float32)
    enorm = en_ref[kb]
    emb2 = emb_ref[...] + emb_ref[...]
    for i in range(_GB):
        zb = z_ref[i]
        s2 = jax.lax.dot_general(emb2, zb, (((1,), (0,)), ((), ())),
                                 preferred_element_type=jnp.float32)
        d = (znorm_ref[i] + enorm) - s2
        bmin = jnp.min(d, axis=0, keepdims=True)
        bidxf = jnp.argmin(d, axis=0).astype(jnp.float32)[None, :]

        @pl.when(kb == 0)
        def _():
            minv_ref[i] = bmin
            mini_ref[i] = bidxf

        @pl.when(kb > 0)
        def _():
            better = bmin < minv_ref[i]
            minv_ref[i] = jnp.where(better, bmin, minv_ref[i])
            mini_ref[i] = jnp.where(better, bidxf + kbase, mini_ref[i])

    @pl.when(kb == _NKB - 1)
    def _():
        for i in range(_GB):
            idx_ref[i] = mini_ref[i].astype(jnp.int32)


_sc_mesh = plsc.VectorSubcoreMesh(core_axis_name="c", subcore_axis_name="s")


@functools.partial(
    pl.kernel,
    mesh=_sc_mesh,
    out_type=jax.ShapeDtypeStruct((_B * _T, _D), jnp.float32),
    scratch_types=[
        pltpu.VMEM((_NCHUNK, _CHUNK), jnp.int32),
        pltpu.VMEM((_RPW, _D), jnp.float32),
        pltpu.SemaphoreType.DMA,
    ],
)
def _sc_gather(idx_hbm, table_hbm, out_hbm, idx_v, rows_v, sem):
    wid = lax.axis_index("s") * 2 + lax.axis_index("c")
    pltpu.sync_copy(idx_hbm.at[wid], idx_v)
    copies = [
        pltpu.async_copy(table_hbm.at[idx_v.at[j]],
                         rows_v.at[pl.ds(j * _CHUNK, _CHUNK)], sem)
        for j in range(_NCHUNK)
    ]
    for cp in copies:
        cp.wait()
    pltpu.sync_copy(rows_v, out_hbm.at[pl.ds(wid * _RPW, _RPW)])


def _transpose_body(in_ref, out_ref):
    out_ref[0] = in_ref[0].T


def kernel(z, embedding):
    idx = pl.pallas_call(
        _argmin_body,
        grid=(_NBG, _NKB),
        in_specs=[pl.BlockSpec((_GB, _D, _T), lambda g, k: (g, 0, 0)),
                  pl.BlockSpec((_KB, _D), lambda g, k: (k, 0))],
        out_specs=pl.BlockSpec((_GB, 1, _T), lambda g, k: (g, 0, 0)),
        out_shape=jax.ShapeDtypeStruct((_B, 1, _T), jnp.int32),
        scratch_shapes=[pltpu.VMEM((_GB, 1, _T), jnp.float32),
                        pltpu.VMEM((_GB, 1, _T), jnp.float32),
                        pltpu.VMEM((_GB, 1, _T), jnp.float32),
                        pltpu.VMEM((_NKB, _KB, 1), jnp.float32)],
    )(z, embedding)
    idx3 = idx.reshape(_NW, _NCHUNK, _CHUNK)
    zq_flat = _sc_gather(idx3, embedding)
    zq = zq_flat.reshape(_B, _T, _D)
    out = pl.pallas_call(
        _transpose_body,
        grid=(_B,),
        in_specs=[pl.BlockSpec((1, _T, _D), lambda b: (b, 0, 0))],
        out_specs=pl.BlockSpec((1, _D, _T), lambda b: (b, 0, 0)),
        out_shape=jax.ShapeDtypeStruct((_B, _D, _T), jnp.float32),
    )(zq)
    return out
